# Initial kernel scaffold; baseline (speedup 1.0000x reference)
import jax
import jax.numpy as jnp
from jax.experimental import pallas as pl


def kernel(x, W0, a_src0, a_dst0, b0, W1, a_src1, a_dst1, b1, att_w, att_b, fc_w, fc_b):
    raise NotImplementedError("write your pallas kernel here")


# speedup vs baseline: 47.0546x; 47.0546x over previous
import functools

import jax
import jax.numpy as jnp
from jax.experimental import pallas as pl

BATCH = 32
SEQ = 512
IN_DIM = 128
HID = 64
HEADS = 4
NUM_CLASSES = 4
NEG_SLOPE = 0.2
FEAT = HEADS * HID

G = 4
ROWS = G * SEQ


def _leaky(v):
    return jnp.where(v >= 0, v, NEG_SLOPE * v)


def _gat_messages(h, a_src_row, a_dst_row, b_row, pos):
    R = h.shape[0]
    p_src = h * a_src_row
    p_dst = h * a_dst_row
    valid_l = pos != 0
    valid_r = pos != (SEQ - 1)
    zero1 = jnp.zeros((1, 1), dtype=h.dtype)
    zeroh = jnp.zeros((1, HID), dtype=h.dtype)
    outs = []
    for head in range(HEADS):
        sl = slice(HID * head, HID * (head + 1))
        a_s = jnp.sum(p_src[:, sl], axis=1, keepdims=True)
        a_d = jnp.sum(p_dst[:, sl], axis=1, keepdims=True)
        a_s_prev = jnp.concatenate([zero1, a_s[: R - 1]], axis=0)
        a_s_next = jnp.concatenate([a_s[1:], zero1], axis=0)
        e_l = _leaky(a_s_prev + a_d)
        e_r = _leaky(a_s_next + a_d)
        m = jnp.maximum(jnp.where(valid_l, e_l, -1e30),
                        jnp.where(valid_r, e_r, -1e30))
        w_l = jnp.where(valid_l, jnp.exp(e_l - m), 0.0)
        w_r = jnp.where(valid_r, jnp.exp(e_r - m), 0.0)
        denom = w_l + w_r + 1e-16
        al = w_l / denom
        ar = w_r / denom
        hh = h[:, sl]
        h_prev = jnp.concatenate([zeroh, hh[: R - 1]], axis=0)
        h_next = jnp.concatenate([hh[1:], zeroh], axis=0)
        outs.append(al * h_prev + ar * h_next)
    return jnp.concatenate(outs, axis=1) + b_row


def _fwd(x_ref, w0_ref, as0_ref, ad0_ref, b0_ref,
         w1_ref, as1_ref, ad1_ref, b1_ref,
         attw_ref, fcw_ref, fcb_ref, out_ref):
    x = x_ref[...]
    pos = jax.lax.broadcasted_iota(jnp.int32, (ROWS, 1), 0) % SEQ

    h = jnp.dot(x, w0_ref[...], preferred_element_type=jnp.float32)
    h = _gat_messages(h, as0_ref[...], ad0_ref[...], b0_ref[...], pos)
    h = jnp.maximum(h, 0.0)

    h = jnp.dot(h, w1_ref[...], preferred_element_type=jnp.float32)
    h = _gat_messages(h, as1_ref[...], ad1_ref[...], b1_ref[...], pos)
    h = jnp.maximum(h, 0.0)

    scores = jnp.sum(h * attw_ref[...], axis=1, keepdims=True)
    fcw = fcw_ref[...]
    fcb = fcb_ref[...]
    for s in range(G):
        sl = slice(s * SEQ, (s + 1) * SEQ)
        sc = scores[sl]
        w = jnp.exp(sc - jnp.max(sc))
        w = w / jnp.sum(w)
        pooled = jnp.sum(h[sl] * w, axis=0, keepdims=True)
        out_ref[0, s : s + 1, :] = (
            jnp.dot(pooled, fcw, preferred_element_type=jnp.float32) + fcb
        )


@functools.partial(jax.jit, static_argnames=())
def kernel(x, W0, a_src0, a_dst0, b0, W1, a_src1, a_dst1, b1,
           att_w, att_b, fc_w, fc_b):
    xf = x.reshape(BATCH * SEQ, IN_DIM)
    as0 = a_src0.reshape(1, FEAT)
    ad0 = a_dst0.reshape(1, FEAT)
    as1 = a_src1.reshape(1, FEAT)
    ad1 = a_dst1.reshape(1, FEAT)
    b0r = b0.reshape(1, FEAT)
    b1r = b1.reshape(1, FEAT)
    attw = att_w.reshape(1, FEAT)
    fcw = jnp.zeros((FEAT, 128), dtype=fc_w.dtype).at[:, :NUM_CLASSES].set(fc_w)
    fcb = jnp.zeros((1, 128), dtype=fc_b.dtype).at[0, :NUM_CLASSES].set(fc_b)

    full = lambda shape: pl.BlockSpec(shape, lambda i: (0, 0))
    out = pl.pallas_call(
        _fwd,
        grid=(BATCH // G,),
        in_specs=[
            pl.BlockSpec((ROWS, IN_DIM), lambda i: (i, 0)),
            full((IN_DIM, FEAT)),
            full((1, FEAT)), full((1, FEAT)), full((1, FEAT)),
            full((FEAT, FEAT)),
            full((1, FEAT)), full((1, FEAT)), full((1, FEAT)),
            full((1, FEAT)),
            full((FEAT, 128)),
            full((1, 128)),
        ],
        out_specs=pl.BlockSpec((1, G, 128), lambda i: (i, 0, 0)),
        out_shape=jax.ShapeDtypeStruct((BATCH // G, G, 128), jnp.float32),
    )(xf, W0, as0, ad0, b0r, W1, as1, ad1, b1r, attw, fcw, fcb)
    return out.reshape(BATCH, 128)[:, :NUM_CLASSES]


# speedup vs baseline: 143.0626x; 3.0404x over previous
import functools

import jax
import jax.numpy as jnp
from jax.experimental import pallas as pl

BATCH = 32
SEQ = 512
IN_DIM = 128
HID = 64
HEADS = 4
NUM_CLASSES = 4
NEG_SLOPE = 0.2
FEAT = HEADS * HID

G = 4
ROWS = G * SEQ


def _leaky(v):
    return jnp.maximum(v, NEG_SLOPE * v)


def _gat_messages(h, A_src, A_dst, b_row, valid_l, valid_r):
    R = h.shape[0]
    s_src = jnp.dot(h, A_src, preferred_element_type=jnp.float32)
    s_dst = jnp.dot(h, A_dst, preferred_element_type=jnp.float32)
    zrow = jnp.zeros((1, FEAT), dtype=h.dtype)
    s_src_prev = jnp.concatenate([zrow, s_src[: R - 1]], axis=0)
    s_src_next = jnp.concatenate([s_src[1:], zrow], axis=0)
    e_l = _leaky(s_src_prev + s_dst)
    e_r = _leaky(s_src_next + s_dst)
    al = jax.nn.sigmoid(e_l - e_r)
    al = jnp.where(valid_r, al, 1.0)
    al = jnp.where(valid_l, al, 0.0)
    ar = 1.0 - al
    h_prev = jnp.concatenate([zrow, h[: R - 1]], axis=0)
    h_next = jnp.concatenate([h[1:], zrow], axis=0)
    return al * h_prev + ar * h_next + b_row


def _fwd(x_ref, w0_ref, as0_ref, ad0_ref, b0_ref,
         w1_ref, as1_ref, ad1_ref, b1_ref,
         attw_ref, fcw_ref, fcb_ref, out_ref):
    x = x_ref[...]
    pos = jax.lax.broadcasted_iota(jnp.int32, (ROWS, FEAT), 0) % SEQ
    valid_l = pos != 0
    valid_r = pos != (SEQ - 1)

    h = jnp.dot(x, w0_ref[...], preferred_element_type=jnp.float32)
    h = _gat_messages(h, as0_ref[...], ad0_ref[...], b0_ref[...],
                      valid_l, valid_r)
    h = jnp.maximum(h, 0.0)

    h = jnp.dot(h, w1_ref[...], preferred_element_type=jnp.float32)
    h = _gat_messages(h, as1_ref[...], ad1_ref[...], b1_ref[...],
                      valid_l, valid_r)
    h = jnp.maximum(h, 0.0)

    scores = jnp.sum(h * attw_ref[...], axis=1, keepdims=True)
    fcw = fcw_ref[...]
    fcb = fcb_ref[...]
    for s in range(G):
        sl = slice(s * SEQ, (s + 1) * SEQ)
        sc = scores[sl]
        w = jnp.exp(sc - jnp.max(sc))
        w = w / jnp.sum(w)
        pooled = jnp.sum(h[sl] * w, axis=0, keepdims=True)
        out_ref[0, s : s + 1, :] = (
            jnp.dot(pooled, fcw, preferred_element_type=jnp.float32) + fcb
        )


@functools.partial(jax.jit, static_argnames=())
def kernel(x, W0, a_src0, a_dst0, b0, W1, a_src1, a_dst1, b1,
           att_w, att_b, fc_w, fc_b):
    xf = x.reshape(BATCH * SEQ, IN_DIM)
    blk = jnp.kron(jnp.eye(HEADS, dtype=x.dtype),
                   jnp.ones((HID, HID), dtype=x.dtype))
    as0 = a_src0.reshape(FEAT, 1) * blk
    ad0 = a_dst0.reshape(FEAT, 1) * blk
    as1 = a_src1.reshape(FEAT, 1) * blk
    ad1 = a_dst1.reshape(FEAT, 1) * blk
    b0r = b0.reshape(1, FEAT)
    b1r = b1.reshape(1, FEAT)
    attw = att_w.reshape(1, FEAT)
    fcw = jnp.zeros((FEAT, 128), dtype=fc_w.dtype).at[:, :NUM_CLASSES].set(fc_w)
    fcb = jnp.zeros((1, 128), dtype=fc_b.dtype).at[0, :NUM_CLASSES].set(fc_b)

    full = lambda shape: pl.BlockSpec(shape, lambda i: (0, 0))
    out = pl.pallas_call(
        _fwd,
        grid=(BATCH // G,),
        in_specs=[
            pl.BlockSpec((ROWS, IN_DIM), lambda i: (i, 0)),
            full((IN_DIM, FEAT)),
            full((FEAT, FEAT)), full((FEAT, FEAT)), full((1, FEAT)),
            full((FEAT, FEAT)),
            full((FEAT, FEAT)), full((FEAT, FEAT)), full((1, FEAT)),
            full((1, FEAT)),
            full((FEAT, 128)),
            full((1, 128)),
        ],
        out_specs=pl.BlockSpec((1, G, 128), lambda i: (i, 0, 0)),
        out_shape=jax.ShapeDtypeStruct((BATCH // G, G, 128), jnp.float32),
    )(xf, W0, as0, ad0, b0r, W1, as1, ad1, b1r, attw, fcw, fcb)
    return out.reshape(BATCH, 128)[:, :NUM_CLASSES]


# speedup vs baseline: 149.1076x; 1.0423x over previous
import functools

import jax
import jax.numpy as jnp
from jax.experimental import pallas as pl

BATCH = 32
SEQ = 512
IN_DIM = 128
HID = 64
HEADS = 4
NUM_CLASSES = 4
NEG_SLOPE = 0.2
FEAT = HEADS * HID

G = 8
ROWS = G * SEQ


def _leaky(v):
    return jnp.maximum(v, NEG_SLOPE * v)


def _gat_messages(h, A_src, A_dst, b_row, valid_l, valid_r):
    R = h.shape[0]
    s_src = jnp.dot(h, A_src, preferred_element_type=jnp.float32)
    s_dst = jnp.dot(h, A_dst, preferred_element_type=jnp.float32)
    zrow = jnp.zeros((1, FEAT), dtype=h.dtype)
    s_src_prev = jnp.concatenate([zrow, s_src[: R - 1]], axis=0)
    s_src_next = jnp.concatenate([s_src[1:], zrow], axis=0)
    e_l = _leaky(s_src_prev + s_dst)
    e_r = _leaky(s_src_next + s_dst)
    al = jax.nn.sigmoid(e_l - e_r)
    al = jnp.where(valid_r, al, 1.0)
    al = jnp.where(valid_l, al, 0.0)
    ar = 1.0 - al
    h_prev = jnp.concatenate([zrow, h[: R - 1]], axis=0)
    h_next = jnp.concatenate([h[1:], zrow], axis=0)
    return al * h_prev + ar * h_next + b_row


def _fwd(x_ref, w0_ref, as0_ref, ad0_ref, b0_ref,
         w1_ref, as1_ref, ad1_ref, b1_ref,
         attw_ref, fcw_ref, fcb_ref, out_ref):
    x = x_ref[...]
    pos = jax.lax.broadcasted_iota(jnp.int32, (ROWS, FEAT), 0) % SEQ
    valid_l = pos != 0
    valid_r = pos != (SEQ - 1)

    h = jnp.dot(x, w0_ref[...], preferred_element_type=jnp.float32)
    h = _gat_messages(h, as0_ref[...], ad0_ref[...], b0_ref[...],
                      valid_l, valid_r)
    h = jnp.maximum(h, 0.0)

    h = jnp.dot(h, w1_ref[...], preferred_element_type=jnp.float32)
    h = _gat_messages(h, as1_ref[...], ad1_ref[...], b1_ref[...],
                      valid_l, valid_r)
    h = jnp.maximum(h, 0.0)

    scores = jnp.sum(h * attw_ref[...], axis=1, keepdims=True)
    fcw = fcw_ref[...]
    fcb = fcb_ref[...]
    for s in range(G):
        sl = slice(s * SEQ, (s + 1) * SEQ)
        sc = scores[sl]
        w = jnp.exp(sc - jnp.max(sc))
        w = w / jnp.sum(w)
        pooled = jnp.sum(h[sl] * w, axis=0, keepdims=True)
        out_ref[0, s : s + 1, :] = (
            jnp.dot(pooled, fcw, preferred_element_type=jnp.float32) + fcb
        )


@functools.partial(jax.jit, static_argnames=())
def kernel(x, W0, a_src0, a_dst0, b0, W1, a_src1, a_dst1, b1,
           att_w, att_b, fc_w, fc_b):
    xf = x.reshape(BATCH * SEQ, IN_DIM)
    blk = jnp.kron(jnp.eye(HEADS, dtype=x.dtype),
                   jnp.ones((HID, HID), dtype=x.dtype))
    as0 = a_src0.reshape(FEAT, 1) * blk
    ad0 = a_dst0.reshape(FEAT, 1) * blk
    as1 = a_src1.reshape(FEAT, 1) * blk
    ad1 = a_dst1.reshape(FEAT, 1) * blk
    b0r = b0.reshape(1, FEAT)
    b1r = b1.reshape(1, FEAT)
    attw = att_w.reshape(1, FEAT)
    fcw = jnp.zeros((FEAT, 128), dtype=fc_w.dtype).at[:, :NUM_CLASSES].set(fc_w)
    fcb = jnp.zeros((1, 128), dtype=fc_b.dtype).at[0, :NUM_CLASSES].set(fc_b)

    full = lambda shape: pl.BlockSpec(shape, lambda i: (0, 0))
    out = pl.pallas_call(
        _fwd,
        grid=(BATCH // G,),
        in_specs=[
            pl.BlockSpec((ROWS, IN_DIM), lambda i: (i, 0)),
            full((IN_DIM, FEAT)),
            full((FEAT, FEAT)), full((FEAT, FEAT)), full((1, FEAT)),
            full((FEAT, FEAT)),
            full((FEAT, FEAT)), full((FEAT, FEAT)), full((1, FEAT)),
            full((1, FEAT)),
            full((FEAT, 128)),
            full((1, 128)),
        ],
        out_specs=pl.BlockSpec((1, G, 128), lambda i: (i, 0, 0)),
        out_shape=jax.ShapeDtypeStruct((BATCH // G, G, 128), jnp.float32),
    )(xf, W0, as0, ad0, b0r, W1, as1, ad1, b1r, attw, fcw, fcb)
    return out.reshape(BATCH, 128)[:, :NUM_CLASSES]


# speedup vs baseline: 150.8048x; 1.0114x over previous
import functools

import jax
import jax.numpy as jnp
from jax.experimental import pallas as pl

BATCH = 32
SEQ = 512
IN_DIM = 128
HID = 64
HEADS = 4
NUM_CLASSES = 4
NEG_SLOPE = 0.2
FEAT = HEADS * HID

G = 16
ROWS = G * SEQ


def _leaky(v):
    return jnp.maximum(v, NEG_SLOPE * v)


def _gat_messages(h, A_src, A_dst, b_row, valid_l, valid_r):
    R = h.shape[0]
    s_src = jnp.dot(h, A_src, preferred_element_type=jnp.float32)
    s_dst = jnp.dot(h, A_dst, preferred_element_type=jnp.float32)
    zrow = jnp.zeros((1, FEAT), dtype=h.dtype)
    s_src_prev = jnp.concatenate([zrow, s_src[: R - 1]], axis=0)
    s_src_next = jnp.concatenate([s_src[1:], zrow], axis=0)
    e_l = _leaky(s_src_prev + s_dst)
    e_r = _leaky(s_src_next + s_dst)
    al = jax.nn.sigmoid(e_l - e_r)
    al = jnp.where(valid_r, al, 1.0)
    al = jnp.where(valid_l, al, 0.0)
    ar = 1.0 - al
    h_prev = jnp.concatenate([zrow, h[: R - 1]], axis=0)
    h_next = jnp.concatenate([h[1:], zrow], axis=0)
    return al * h_prev + ar * h_next + b_row


def _fwd(x_ref, w0_ref, as0_ref, ad0_ref, b0_ref,
         w1_ref, as1_ref, ad1_ref, b1_ref,
         attw_ref, fcw_ref, fcb_ref, out_ref):
    x = x_ref[...]
    pos = jax.lax.broadcasted_iota(jnp.int32, (ROWS, FEAT), 0) % SEQ
    valid_l = pos != 0
    valid_r = pos != (SEQ - 1)

    h = jnp.dot(x, w0_ref[...], preferred_element_type=jnp.float32)
    h = _gat_messages(h, as0_ref[...], ad0_ref[...], b0_ref[...],
                      valid_l, valid_r)
    h = jnp.maximum(h, 0.0)

    h = jnp.dot(h, w1_ref[...], preferred_element_type=jnp.float32)
    h = _gat_messages(h, as1_ref[...], ad1_ref[...], b1_ref[...],
                      valid_l, valid_r)
    h = jnp.maximum(h, 0.0)

    scores = jnp.sum(h * attw_ref[...], axis=1, keepdims=True)
    fcw = fcw_ref[...]
    fcb = fcb_ref[...]
    for s in range(G):
        sl = slice(s * SEQ, (s + 1) * SEQ)
        sc = scores[sl]
        w = jnp.exp(sc - jnp.max(sc))
        w = w / jnp.sum(w)
        pooled = jnp.sum(h[sl] * w, axis=0, keepdims=True)
        out_ref[0, s : s + 1, :] = (
            jnp.dot(pooled, fcw, preferred_element_type=jnp.float32) + fcb
        )


@functools.partial(jax.jit, static_argnames=())
def kernel(x, W0, a_src0, a_dst0, b0, W1, a_src1, a_dst1, b1,
           att_w, att_b, fc_w, fc_b):
    xf = x.reshape(BATCH * SEQ, IN_DIM)
    blk = jnp.kron(jnp.eye(HEADS, dtype=x.dtype),
                   jnp.ones((HID, HID), dtype=x.dtype))
    as0 = a_src0.reshape(FEAT, 1) * blk
    ad0 = a_dst0.reshape(FEAT, 1) * blk
    as1 = a_src1.reshape(FEAT, 1) * blk
    ad1 = a_dst1.reshape(FEAT, 1) * blk
    b0r = b0.reshape(1, FEAT)
    b1r = b1.reshape(1, FEAT)
    attw = att_w.reshape(1, FEAT)
    fcw = jnp.zeros((FEAT, 128), dtype=fc_w.dtype).at[:, :NUM_CLASSES].set(fc_w)
    fcb = jnp.zeros((1, 128), dtype=fc_b.dtype).at[0, :NUM_CLASSES].set(fc_b)

    full = lambda shape: pl.BlockSpec(shape, lambda i: (0, 0))
    out = pl.pallas_call(
        _fwd,
        grid=(BATCH // G,),
        in_specs=[
            pl.BlockSpec((ROWS, IN_DIM), lambda i: (i, 0)),
            full((IN_DIM, FEAT)),
            full((FEAT, FEAT)), full((FEAT, FEAT)), full((1, FEAT)),
            full((FEAT, FEAT)),
            full((FEAT, FEAT)), full((FEAT, FEAT)), full((1, FEAT)),
            full((1, FEAT)),
            full((FEAT, 128)),
            full((1, 128)),
        ],
        out_specs=pl.BlockSpec((1, G, 128), lambda i: (i, 0, 0)),
        out_shape=jax.ShapeDtypeStruct((BATCH // G, G, 128), jnp.float32),
    )(xf, W0, as0, ad0, b0r, W1, as1, ad1, b1r, attw, fcw, fcb)
    return out.reshape(BATCH, 128)[:, :NUM_CLASSES]


# speedup vs baseline: 171.1239x; 1.1347x over previous
import functools

import jax
import jax.numpy as jnp
from jax.experimental import pallas as pl

BATCH = 32
SEQ = 512
IN_DIM = 128
HID = 64
HEADS = 4
NUM_CLASSES = 4
NEG_SLOPE = 0.2
FEAT = HEADS * HID

G = 16
ROWS = G * SEQ


def _leaky(v):
    return jnp.maximum(v, NEG_SLOPE * v)


def _gat_messages(h, A_src, A_dst, bcast, b_row, valid_l, valid_r):
    R = h.shape[0]
    s_src = jnp.dot(h, A_src, preferred_element_type=jnp.float32)
    s_dst = jnp.dot(h, A_dst, preferred_element_type=jnp.float32)
    znar = jnp.zeros((1, 128), dtype=h.dtype)
    s_src_prev = jnp.concatenate([znar, s_src[: R - 1]], axis=0)
    s_src_next = jnp.concatenate([s_src[1:], znar], axis=0)
    e_l = _leaky(s_src_prev + s_dst)
    e_r = _leaky(s_src_next + s_dst)
    al = jax.nn.sigmoid(e_l - e_r)
    al = jnp.where(valid_r, al, 1.0)
    al = jnp.where(valid_l, al, 0.0)
    al = jnp.dot(al, bcast, preferred_element_type=jnp.float32)
    zrow = jnp.zeros((1, FEAT), dtype=h.dtype)
    h_prev = jnp.concatenate([zrow, h[: R - 1]], axis=0)
    h_next = jnp.concatenate([h[1:], zrow], axis=0)
    return h_next + al * (h_prev - h_next) + b_row


def _fwd(x_ref, w0_ref, as0_ref, ad0_ref, b0_ref,
         w1_ref, as1_ref, ad1_ref, b1_ref,
         attw_ref, fcw_ref, fcb_ref, bcast_ref, out_ref):
    x = x_ref[...]
    pos = jax.lax.broadcasted_iota(jnp.int32, (ROWS, 128), 0) % SEQ
    valid_l = pos != 0
    valid_r = pos != (SEQ - 1)
    bcast = bcast_ref[...]

    h = jnp.dot(x, w0_ref[...], preferred_element_type=jnp.float32)
    h = _gat_messages(h, as0_ref[...], ad0_ref[...], bcast, b0_ref[...],
                      valid_l, valid_r)
    h = jnp.maximum(h, 0.0)

    h = jnp.dot(h, w1_ref[...], preferred_element_type=jnp.float32)
    h = _gat_messages(h, as1_ref[...], ad1_ref[...], bcast, b1_ref[...],
                      valid_l, valid_r)
    h = jnp.maximum(h, 0.0)

    scores = jnp.sum(h * attw_ref[...], axis=1, keepdims=True)
    fcw = fcw_ref[...]
    fcb = fcb_ref[...]
    for s in range(G):
        sl = slice(s * SEQ, (s + 1) * SEQ)
        sc = scores[sl]
        w = jnp.exp(sc - jnp.max(sc))
        w = w / jnp.sum(w)
        pooled = jnp.sum(h[sl] * w, axis=0, keepdims=True)
        out_ref[0, s : s + 1, :] = (
            jnp.dot(pooled, fcw, preferred_element_type=jnp.float32) + fcb
        )


@functools.partial(jax.jit, static_argnames=())
def kernel(x, W0, a_src0, a_dst0, b0, W1, a_src1, a_dst1, b1,
           att_w, att_b, fc_w, fc_b):
    xf = x.reshape(BATCH * SEQ, IN_DIM)
    hsel = jnp.kron(jnp.eye(HEADS, dtype=x.dtype),
                    jnp.ones((HID, 1), dtype=x.dtype))
    padn = jnp.zeros((FEAT, 128 - HEADS), dtype=x.dtype)
    as0 = jnp.concatenate([a_src0.reshape(FEAT, 1) * hsel, padn], axis=1)
    ad0 = jnp.concatenate([a_dst0.reshape(FEAT, 1) * hsel, padn], axis=1)
    as1 = jnp.concatenate([a_src1.reshape(FEAT, 1) * hsel, padn], axis=1)
    ad1 = jnp.concatenate([a_dst1.reshape(FEAT, 1) * hsel, padn], axis=1)
    bcast = jnp.concatenate([
        jnp.kron(jnp.eye(HEADS, dtype=x.dtype),
                 jnp.ones((1, HID), dtype=x.dtype)),
        jnp.zeros((128 - HEADS, FEAT), dtype=x.dtype)], axis=0)
    b0r = b0.reshape(1, FEAT)
    b1r = b1.reshape(1, FEAT)
    attw = att_w.reshape(1, FEAT)
    fcw = jnp.zeros((FEAT, 128), dtype=fc_w.dtype).at[:, :NUM_CLASSES].set(fc_w)
    fcb = jnp.zeros((1, 128), dtype=fc_b.dtype).at[0, :NUM_CLASSES].set(fc_b)

    full = lambda shape: pl.BlockSpec(shape, lambda i: (0, 0))
    out = pl.pallas_call(
        _fwd,
        grid=(BATCH // G,),
        in_specs=[
            pl.BlockSpec((ROWS, IN_DIM), lambda i: (i, 0)),
            full((IN_DIM, FEAT)),
            full((FEAT, 128)), full((FEAT, 128)), full((1, FEAT)),
            full((FEAT, FEAT)),
            full((FEAT, 128)), full((FEAT, 128)), full((1, FEAT)),
            full((1, FEAT)),
            full((FEAT, 128)),
            full((1, 128)),
            full((128, FEAT)),
        ],
        out_specs=pl.BlockSpec((1, G, 128), lambda i: (i, 0, 0)),
        out_shape=jax.ShapeDtypeStruct((BATCH // G, G, 128), jnp.float32),
    )(xf, W0, as0, ad0, b0r, W1, as1, ad1, b1r, attw, fcw, fcb, bcast)
    return out.reshape(BATCH, 128)[:, :NUM_CLASSES]


# speedup vs baseline: 178.2871x; 1.0419x over previous
import functools

import jax
import jax.numpy as jnp
from jax.experimental import pallas as pl

BATCH = 32
SEQ = 512
IN_DIM = 128
HID = 64
HEADS = 4
NUM_CLASSES = 4
NEG_SLOPE = 0.2
FEAT = HEADS * HID

G = 16
ROWS = G * SEQ


def _leaky(v):
    return jnp.maximum(v, NEG_SLOPE * v)


def _logit_mat(a_col):
    k = jax.lax.broadcasted_iota(jnp.int32, (FEAT, 128), 0)
    j = jax.lax.broadcasted_iota(jnp.int32, (FEAT, 128), 1)
    return jnp.where(j == k // HID, a_col, 0.0)


def _bcast_mat(dtype):
    p = jax.lax.broadcasted_iota(jnp.int32, (128, FEAT), 0)
    j = jax.lax.broadcasted_iota(jnp.int32, (128, FEAT), 1)
    return (p == j // HID).astype(dtype)


def _gat_messages(h, a_src, a_dst, bcast, b_row, valid_l, valid_r):
    R = h.shape[0]
    s_src = jnp.dot(h, _logit_mat(a_src), preferred_element_type=jnp.float32)
    s_dst = jnp.dot(h, _logit_mat(a_dst), preferred_element_type=jnp.float32)
    znar = jnp.zeros((1, 128), dtype=h.dtype)
    s_src_prev = jnp.concatenate([znar, s_src[: R - 1]], axis=0)
    s_src_next = jnp.concatenate([s_src[1:], znar], axis=0)
    e_l = _leaky(s_src_prev + s_dst)
    e_r = _leaky(s_src_next + s_dst)
    al = jax.nn.sigmoid(e_l - e_r)
    al = jnp.where(valid_r, al, 1.0)
    al = jnp.where(valid_l, al, 0.0)
    al = jnp.dot(al, bcast, preferred_element_type=jnp.float32)
    zrow = jnp.zeros((1, FEAT), dtype=h.dtype)
    h_prev = jnp.concatenate([zrow, h[: R - 1]], axis=0)
    h_next = jnp.concatenate([h[1:], zrow], axis=0)
    return h_next + al * (h_prev - h_next) + b_row


def _fwd(x_ref, w0_ref, as0_ref, ad0_ref, b0_ref,
         w1_ref, as1_ref, ad1_ref, b1_ref,
         attw_ref, fcw_ref, fcb_ref, out_ref):
    x = x_ref[...]
    pos = jax.lax.broadcasted_iota(jnp.int32, (ROWS, 128), 0) % SEQ
    valid_l = pos != 0
    valid_r = pos != (SEQ - 1)
    bcast = _bcast_mat(x.dtype)

    h = jnp.dot(x, w0_ref[...], preferred_element_type=jnp.float32)
    h = _gat_messages(h, as0_ref[...], ad0_ref[...], bcast, b0_ref[...],
                      valid_l, valid_r)
    h = jnp.maximum(h, 0.0)

    h = jnp.dot(h, w1_ref[...], preferred_element_type=jnp.float32)
    h = _gat_messages(h, as1_ref[...], ad1_ref[...], bcast, b1_ref[...],
                      valid_l, valid_r)
    h = jnp.maximum(h, 0.0)

    scores = jnp.sum(h * attw_ref[...], axis=1, keepdims=True)
    fcw = fcw_ref[...]
    fcb = fcb_ref[...]
    zpad = jnp.zeros((1, 128 - NUM_CLASSES), dtype=x.dtype)
    for s in range(G):
        sl = slice(s * SEQ, (s + 1) * SEQ)
        sc = scores[sl]
        w = jnp.exp(sc - jnp.max(sc))
        w = w / jnp.sum(w)
        pooled = jnp.sum(h[sl] * w, axis=0, keepdims=True)
        res = jnp.dot(pooled, fcw, preferred_element_type=jnp.float32) + fcb
        out_ref[0, s : s + 1, :] = jnp.concatenate([res, zpad], axis=1)


@functools.partial(jax.jit, static_argnames=())
def kernel(x, W0, a_src0, a_dst0, b0, W1, a_src1, a_dst1, b1,
           att_w, att_b, fc_w, fc_b):
    xf = x.reshape(BATCH * SEQ, IN_DIM)
    full = lambda shape: pl.BlockSpec(shape, lambda i: (0, 0))
    out = pl.pallas_call(
        _fwd,
        grid=(BATCH // G,),
        in_specs=[
            pl.BlockSpec((ROWS, IN_DIM), lambda i: (i, 0)),
            full((IN_DIM, FEAT)),
            full((FEAT, 1)), full((FEAT, 1)), full((1, FEAT)),
            full((FEAT, FEAT)),
            full((FEAT, 1)), full((FEAT, 1)), full((1, FEAT)),
            full((1, FEAT)),
            full((FEAT, NUM_CLASSES)),
            full((1, NUM_CLASSES)),
        ],
        out_specs=pl.BlockSpec((1, G, 128), lambda i: (i, 0, 0)),
        out_shape=jax.ShapeDtypeStruct((BATCH // G, G, 128), jnp.float32),
    )(xf, W0,
      a_src0.reshape(FEAT, 1), a_dst0.reshape(FEAT, 1), b0.reshape(1, FEAT),
      W1,
      a_src1.reshape(FEAT, 1), a_dst1.reshape(FEAT, 1), b1.reshape(1, FEAT),
      att_w.reshape(1, FEAT),
      fc_w, fc_b.reshape(1, NUM_CLASSES))
    return out.reshape(BATCH, 128)[:, :NUM_CLASSES]
